# Initial kernel scaffold; baseline (speedup 1.0000x reference)
import jax
import jax.numpy as jnp
from jax.experimental import pallas as pl


def kernel(x, edge_index, W1, b1, W2, b2, temp):
    raise NotImplementedError("write your pallas kernel here")


# speedup vs baseline: 12.6993x; 12.6993x over previous
import functools
import math

import jax
import jax.numpy as jnp
import numpy as np
from jax import lax
from jax.experimental import pallas as pl
from jax.experimental.pallas import tpu as pltpu
from jax.experimental.pallas import tpu_sc as plsc

N = 10000
D = 128
K = 10
E = 320000

NC = 2
NS = 16
L = 16
FH = D // NC
BLK = 64
GB = 8
NG = 40
NB = GB * NG
EP = NS * NB * BLK
NP_ = 10240
STRIPE = NP_ // NS
CH = 64
NCH = STRIPE // CH
DUMMY = N
MB = 1000

_BM = np.zeros((K + 1, K + 1), np.float64)
for _j in range(K + 1):
    _p1 = np.array([math.comb(_j, m) * (-1) ** m for m in range(_j + 1)], np.float64)
    _p2 = np.array([math.comb(K - _j, m) for m in range(K - _j + 1)], np.float64)
    _BM[_j, :] = np.convolve(_p1, _p2) * (math.comb(K, _j) / 2.0 ** K)
_BM = _BM.astype(np.float32)


def _mlp_body(x_ref, w1t_ref, b1_ref, w2t_ref, b2_ref, o_ref):
    h1 = jnp.dot(x_ref[...], w1t_ref[...], preferred_element_type=jnp.float32)
    h1 = jnp.maximum(h1 + b1_ref[...], 0.0)
    h2 = jnp.dot(h1, w2t_ref[...], preferred_element_type=jnp.float32)
    o_ref[...] = h2 + b2_ref[...]


def _lsm_body(x_ref, o_ref):
    v = x_ref[...]
    y = v - jnp.max(v, axis=1, keepdims=True)
    o_ref[...] = y - jnp.log(jnp.sum(jnp.exp(y), axis=1, keepdims=True))


def _mlp(x, w1t, b1, w2t, b2):
    return pl.pallas_call(
        _mlp_body,
        grid=(N // MB,),
        in_specs=[
            pl.BlockSpec((MB, D), lambda i: (i, 0)),
            pl.BlockSpec((D, D), lambda i: (0, 0)),
            pl.BlockSpec((1, D), lambda i: (0, 0)),
            pl.BlockSpec((D, D), lambda i: (0, 0)),
            pl.BlockSpec((1, D), lambda i: (0, 0)),
        ],
        out_specs=pl.BlockSpec((MB, D), lambda i: (i, 0)),
        out_shape=jax.ShapeDtypeStruct((N, D), jnp.float32),
    )(x, w1t, b1, w2t, b2)


def _log_softmax(h):
    return pl.pallas_call(
        _lsm_body,
        grid=(N // MB,),
        in_specs=[pl.BlockSpec((MB, D), lambda i: (i, 0))],
        out_specs=pl.BlockSpec((MB, D), lambda i: (i, 0)),
        out_shape=jax.ShapeDtypeStruct((N, D), jnp.float32),
    )(h)


_mesh = plsc.VectorSubcoreMesh(core_axis_name="c", subcore_axis_name="s")


@functools.partial(
    pl.kernel,
    out_type=jax.ShapeDtypeStruct((NP_, L), jnp.float32),
    mesh=_mesh,
    scratch_types=[
        pltpu.VMEM((GB // 2, 2 * BLK), jnp.int32),
        pltpu.VMEM((GB, BLK), jnp.int32),
        pltpu.VMEM((BLK, L), jnp.float32),
        pltpu.VMEM((CH, L), jnp.float32),
        pltpu.VMEM_SHARED((NP_, L), jnp.float32),
    ],
)
def _deg_sc(ei3, out, eibuf, rowsg, onesv, stagev, deg_sp):
    c = lax.axis_index("c")
    s_id = lax.axis_index("s")
    node0 = s_id * STRIPE

    def _ones_body(r, carry):
        onesv[r, :] = jnp.full((L,), 1.0, jnp.float32)
        return carry

    lax.fori_loop(0, BLK, _ones_body, 0)

    def _zero_body(r, carry):
        stagev[r, :] = jnp.zeros((L,), jnp.float32)
        return carry

    lax.fori_loop(0, CH, _zero_body, 0)

    for ch in range(NCH):
        pltpu.sync_copy(stagev, deg_sp.at[pl.ds(node0 + ch * CH, CH)])
    plsc.subcore_barrier()

    def _group_body(g, carry):
        pltpu.sync_copy(ei3.at[s_id, pl.ds(g * (GB // 2), GB // 2)], eibuf)

        def _unpack_body(j, carry2):
            for h2 in range(2):
                for f in range(BLK // L):
                    fs = pl.ds(h2 * BLK + f * L, L)
                    v = lax.bitwise_and(
                        eibuf[j, fs], jnp.full((L,), 0xFFFF, jnp.int32))
                    rowsg[2 * j + h2, pl.ds(f * L, L)] = jnp.minimum(
                        v, jnp.full((L,), NP_ - 1, jnp.int32))
            return carry2

        lax.fori_loop(0, GB // 2, _unpack_body, 0)
        for b in range(GB):
            pltpu.sync_copy(onesv, deg_sp.at[rowsg.at[b]], add=True)
        return carry

    lax.fori_loop(0, NG, _group_body, 0)
    plsc.subcore_barrier()

    @pl.when(c == 0)
    def _():
        for ch in range(NCH):
            sl = pl.ds(node0 + ch * CH, CH)
            pltpu.sync_copy(deg_sp.at[sl], stagev)
            pltpu.sync_copy(stagev, out.at[sl])


@functools.partial(
    pl.kernel,
    out_type=jax.ShapeDtypeStruct((NC * NP_, 2 * FH), jnp.float32),
    mesh=_mesh,
    scratch_types=[
        pltpu.VMEM((GB // 2, 2 * BLK), jnp.int32),
        pltpu.VMEM((GB, BLK), jnp.int32),
        pltpu.VMEM((GB, BLK), jnp.int32),
        pltpu.VMEM((2, BLK, 2 * FH), jnp.float32),
        pltpu.VMEM((CH, 2 * FH), jnp.float32),
        pltpu.VMEM((16, L), jnp.float32),
        pltpu.VMEM_SHARED((NP_, 2 * FH), jnp.float32),
        pltpu.SemaphoreType.DMA,
        pltpu.SemaphoreType.DMA,
    ],
)
def _bern_sc(ei3, hsplit, dcoef, out,
             eibuf, rowsg, colsg, msg, hdv, dv,
             acc_sp, sem0, sem1):
    accv = msg.at[0]
    zerov = msg.at[1]
    c = lax.axis_index("c")
    s_id = lax.axis_index("s")
    node0 = s_id * STRIPE
    soff = c * NP_

    pltpu.sync_copy(dcoef, dv)

    def _scale_chunk(i, mode):
        di = dv[i, :]
        W = 2 * FH // L

        def _zb(r, carry):
            for f in range(W):
                msg[1, r, pl.ds(f * L, L)] = jnp.zeros((L,), jnp.float32)
            return carry

        if mode != "final":
            lax.fori_loop(0, BLK, _zb, 0)

        def _chunk_body(ch, carry):
            base = node0 + ch * CH
            sl = pl.ds(base, CH)
            if mode != "init":
                pltpu.sync_copy(acc_sp.at[sl], accv)
                if mode == "mid":
                    pltpu.sync_copy(zerov, acc_sp.at[sl])
            pltpu.sync_copy(hsplit.at[c, sl], hdv)

            if mode == "init":
                def _body(r, carry2):
                    dvec = hdv[r, pl.ds(FH, L)]
                    for f in range(W):
                        fs = pl.ds(f * L, L)
                        accv[r, fs] = di * dvec * hdv[r, fs]
                    return carry2
            elif mode == "final":
                def _body(r, carry2):
                    dvec = hdv[r, pl.ds(FH, L)]
                    for f in range(W):
                        fs = pl.ds(f * L, L)
                        accv[r, fs] = dvec * accv[r, fs] + di * hdv[r, fs]
                    return carry2
            else:
                def _body(r, carry2):
                    dvec = hdv[r, pl.ds(FH, L)]
                    d2 = hdv[r, pl.ds(FH + L, L)]
                    for f in range(W):
                        fs = pl.ds(f * L, L)
                        accv[r, fs] = d2 * accv[r, fs] + di * dvec * hdv[r, fs]
                    return carry2

            lax.fori_loop(0, CH, _body, 0)
            pltpu.sync_copy(accv, out.at[pl.ds(soff + base, CH)])
            if mode == "init":
                pltpu.sync_copy(zerov, acc_sp.at[sl])
            return carry

        lax.fori_loop(0, NCH, _chunk_body, 0)

    def _edge_phase():
        def _group_body(g, carry):
            pltpu.sync_copy(ei3.at[s_id, pl.ds(g * (GB // 2), GB // 2)], eibuf)

            def _unpack_body(j, carry2):
                cap = jnp.full((L,), NP_ - 1, jnp.int32)
                for h2 in range(2):
                    for f in range(BLK // L):
                        fs = pl.ds(h2 * BLK + f * L, L)
                        packed = eibuf[j, fs]
                        dfs = pl.ds(f * L, L)
                        colsg[2 * j + h2, dfs] = jnp.minimum(
                            lax.shift_right_logical(packed, 16), cap)
                        rowsg[2 * j + h2, dfs] = jnp.minimum(
                            lax.bitwise_and(
                                packed, jnp.full((L,), 0xFFFF, jnp.int32)),
                            cap) + soff
                return carry2

            lax.fori_loop(0, GB // 2, _unpack_body, 0)

            pend = pltpu.async_copy(out.at[rowsg.at[0]], msg.at[0], sem0)
            for b in range(GB):
                nxt = None
                if b + 1 < GB:
                    p = (b + 1) & 1
                    nxt = pltpu.async_copy(
                        out.at[rowsg.at[b + 1]], msg.at[p],
                        sem1 if p else sem0)
                pend.wait()
                pltpu.sync_copy(msg.at[b & 1], acc_sp.at[colsg.at[b]],
                                add=True)
                pend = nxt
            return carry

        lax.fori_loop(0, NG, _group_body, 0)

    _scale_chunk(K, "init")
    plsc.subcore_barrier()
    for i in range(K - 1, 0, -1):
        _edge_phase()
        plsc.subcore_barrier()
        _scale_chunk(i, "mid")
        plsc.subcore_barrier()
    _edge_phase()
    plsc.subcore_barrier()
    _scale_chunk(0, "final")


def kernel(x, edge_index, W1, b1, W2, b2, temp):
    h = _mlp(x, W1.T, b1[None, :], W2.T, b2[None, :])

    ei = jnp.asarray(edge_index, jnp.int32)
    packed = jnp.bitwise_or(ei[0], jnp.left_shift(ei[1], 16))
    pad = jnp.full((EP - E,), DUMMY | (DUMMY << 16), jnp.int32)
    ei3 = jnp.concatenate([packed, pad]).reshape(NS, NB // 2, 2 * BLK)

    degrep = _deg_sc(ei3)
    deg = degrep[:N, 0]
    dis = jnp.where(deg > 0, lax.rsqrt(jnp.where(deg > 0, deg, 1.0)), 0.0)
    disL = jnp.broadcast_to(
        jnp.pad(dis, (0, NP_ - N))[:, None], (NP_, L))

    d = jnp.pad(jax.nn.relu(temp) @ _BM, (0, 16 - (K + 1)))
    dcoef = jnp.broadcast_to(d[:, None], (16, L))

    hp = jnp.pad(h, ((0, NP_ - N), (0, 0)))
    zpad = jnp.zeros((NP_, 2 * L), jnp.float32)
    hsplit = jnp.stack(
        [jnp.concatenate([hp[:, :FH], disL, disL * disL, zpad], axis=1),
         jnp.concatenate([hp[:, FH:], disL, disL * disL, zpad], axis=1)],
        axis=0)

    res = _bern_sc(ei3, hsplit, dcoef)
    prop = jnp.concatenate(
        [res[:N, :FH], res[NP_:NP_ + N, :FH]], axis=1)

    return _log_softmax(prop)


# speedup vs baseline: 13.6718x; 1.0766x over previous
import functools
import math

import jax
import jax.numpy as jnp
import numpy as np
from jax import lax
from jax.experimental import pallas as pl
from jax.experimental.pallas import tpu as pltpu
from jax.experimental.pallas import tpu_sc as plsc

N = 10000
D = 128
K = 10
E = 320000

NC = 2
NS = 16
L = 16
FH = D // NC
BLK = 128
GB = 8
NG = 20
NB = GB * NG
EP = NS * NB * BLK
NP_ = 10240
STRIPE = NP_ // NS
CH = 64
NCH = STRIPE // CH
DUMMY = N
MB = 1000

_BM = np.zeros((K + 1, K + 1), np.float64)
for _j in range(K + 1):
    _p1 = np.array([math.comb(_j, m) * (-1) ** m for m in range(_j + 1)], np.float64)
    _p2 = np.array([math.comb(K - _j, m) for m in range(K - _j + 1)], np.float64)
    _BM[_j, :] = np.convolve(_p1, _p2) * (math.comb(K, _j) / 2.0 ** K)
_BM = _BM.astype(np.float32)


def _mlp_body(x_ref, w1t_ref, b1_ref, w2t_ref, b2_ref, o_ref):
    h1 = jnp.dot(x_ref[...], w1t_ref[...], preferred_element_type=jnp.float32)
    h1 = jnp.maximum(h1 + b1_ref[...], 0.0)
    h2 = jnp.dot(h1, w2t_ref[...], preferred_element_type=jnp.float32)
    o_ref[...] = h2 + b2_ref[...]


def _lsm_body(x_ref, o_ref):
    v = x_ref[...]
    y = v - jnp.max(v, axis=1, keepdims=True)
    o_ref[...] = y - jnp.log(jnp.sum(jnp.exp(y), axis=1, keepdims=True))


def _mlp(x, w1t, b1, w2t, b2):
    return pl.pallas_call(
        _mlp_body,
        grid=(N // MB,),
        in_specs=[
            pl.BlockSpec((MB, D), lambda i: (i, 0)),
            pl.BlockSpec((D, D), lambda i: (0, 0)),
            pl.BlockSpec((1, D), lambda i: (0, 0)),
            pl.BlockSpec((D, D), lambda i: (0, 0)),
            pl.BlockSpec((1, D), lambda i: (0, 0)),
        ],
        out_specs=pl.BlockSpec((MB, D), lambda i: (i, 0)),
        out_shape=jax.ShapeDtypeStruct((N, D), jnp.float32),
    )(x, w1t, b1, w2t, b2)


def _log_softmax(h):
    return pl.pallas_call(
        _lsm_body,
        grid=(N // MB,),
        in_specs=[pl.BlockSpec((MB, D), lambda i: (i, 0))],
        out_specs=pl.BlockSpec((MB, D), lambda i: (i, 0)),
        out_shape=jax.ShapeDtypeStruct((N, D), jnp.float32),
    )(h)


_mesh = plsc.VectorSubcoreMesh(core_axis_name="c", subcore_axis_name="s")


@functools.partial(
    pl.kernel,
    out_type=jax.ShapeDtypeStruct((NP_, L), jnp.float32),
    mesh=_mesh,
    scratch_types=[
        pltpu.VMEM((GB, BLK), jnp.int32),
        pltpu.VMEM((GB, BLK), jnp.int32),
        pltpu.VMEM((BLK, L), jnp.float32),
        pltpu.VMEM((CH, L), jnp.float32),
        pltpu.VMEM_SHARED((NP_, L), jnp.float32),
    ],
)
def _deg_sc(ei3, out, eibuf, rowsg, onesv, stagev, deg_sp):
    c = lax.axis_index("c")
    s_id = lax.axis_index("s")
    node0 = s_id * STRIPE

    def _ones_body(r, carry):
        onesv[r, :] = jnp.full((L,), 1.0, jnp.float32)
        return carry

    lax.fori_loop(0, BLK, _ones_body, 0)

    def _zero_body(r, carry):
        stagev[r, :] = jnp.zeros((L,), jnp.float32)
        return carry

    lax.fori_loop(0, CH, _zero_body, 0)

    for ch in range(NCH):
        pltpu.sync_copy(stagev, deg_sp.at[pl.ds(node0 + ch * CH, CH)])
    plsc.subcore_barrier()

    def _group_body(g, carry):
        pltpu.sync_copy(ei3.at[s_id, pl.ds(g * GB, GB)], eibuf)

        def _unpack_body(j, carry2):
            for f in range(BLK // L):
                fs = pl.ds(f * L, L)
                v = lax.bitwise_and(
                    eibuf[j, fs], jnp.full((L,), 0xFFFF, jnp.int32))
                rowsg[j, fs] = jnp.minimum(
                    v, jnp.full((L,), NP_ - 1, jnp.int32))
            return carry2

        lax.fori_loop(0, GB, _unpack_body, 0)
        for b in range(GB):
            pltpu.sync_copy(onesv, deg_sp.at[rowsg.at[b]], add=True)
        return carry

    lax.fori_loop(0, NG, _group_body, 0)
    plsc.subcore_barrier()

    @pl.when(c == 0)
    def _():
        for ch in range(NCH):
            sl = pl.ds(node0 + ch * CH, CH)
            pltpu.sync_copy(deg_sp.at[sl], stagev)
            pltpu.sync_copy(stagev, out.at[sl])


@functools.partial(
    pl.kernel,
    out_type=jax.ShapeDtypeStruct((NC * NP_, 2 * FH), jnp.float32),
    mesh=_mesh,
    scratch_types=[
        pltpu.VMEM((GB, BLK), jnp.int32),
        pltpu.VMEM((GB, BLK), jnp.int32),
        pltpu.VMEM((GB, BLK), jnp.int32),
        pltpu.VMEM((2, BLK, 2 * FH), jnp.float32),
        pltpu.VMEM((CH, 2 * FH), jnp.float32),
        pltpu.VMEM((16, L), jnp.float32),
        pltpu.VMEM_SHARED((NP_, 2 * FH), jnp.float32),
        pltpu.SemaphoreType.DMA,
        pltpu.SemaphoreType.DMA,
    ],
)
def _bern_sc(ei3, hsplit, dcoef, out,
             eibuf, rowsg, colsg, msg, hdv, dv,
             acc_sp, sem0, sem1):
    accv = msg.at[0, pl.ds(0, CH)]
    zerov = msg.at[1, pl.ds(0, CH)]
    c = lax.axis_index("c")
    s_id = lax.axis_index("s")
    node0 = s_id * STRIPE
    soff = c * NP_

    pltpu.sync_copy(dcoef, dv)

    def _scale_chunk(i, mode):
        di = dv[i, :]
        W = 2 * FH // L

        def _zb(r, carry):
            for f in range(W):
                msg[1, r, pl.ds(f * L, L)] = jnp.zeros((L,), jnp.float32)
            return carry

        if mode != "final":
            lax.fori_loop(0, BLK, _zb, 0)

        def _chunk_body(ch, carry):
            base = node0 + ch * CH
            sl = pl.ds(base, CH)
            if mode != "init":
                pltpu.sync_copy(acc_sp.at[sl], accv)
                if mode == "mid":
                    pltpu.sync_copy(zerov, acc_sp.at[sl])
            pltpu.sync_copy(hsplit.at[c, sl], hdv)

            if mode == "init":
                def _body(r, carry2):
                    dvec = hdv[r, pl.ds(FH, L)]
                    for f in range(W):
                        fs = pl.ds(f * L, L)
                        accv[r, fs] = di * dvec * hdv[r, fs]
                    return carry2
            elif mode == "final":
                def _body(r, carry2):
                    dvec = hdv[r, pl.ds(FH, L)]
                    for f in range(W):
                        fs = pl.ds(f * L, L)
                        accv[r, fs] = dvec * accv[r, fs] + di * hdv[r, fs]
                    return carry2
            else:
                def _body(r, carry2):
                    dvec = hdv[r, pl.ds(FH, L)]
                    d2 = hdv[r, pl.ds(FH + L, L)]
                    for f in range(W):
                        fs = pl.ds(f * L, L)
                        accv[r, fs] = d2 * accv[r, fs] + di * dvec * hdv[r, fs]
                    return carry2

            lax.fori_loop(0, CH, _body, 0)
            pltpu.sync_copy(accv, out.at[pl.ds(soff + base, CH)])
            if mode == "init":
                pltpu.sync_copy(zerov, acc_sp.at[sl])
            return carry

        lax.fori_loop(0, NCH, _chunk_body, 0)

    def _edge_phase():
        def _group_body(g, carry):
            pltpu.sync_copy(ei3.at[s_id, pl.ds(g * GB, GB)], eibuf)

            def _unpack_body(j, carry2):
                cap = jnp.full((L,), NP_ - 1, jnp.int32)
                for f in range(BLK // L):
                    fs = pl.ds(f * L, L)
                    packed = eibuf[j, fs]
                    colsg[j, fs] = jnp.minimum(
                        lax.shift_right_logical(packed, 16), cap)
                    rowsg[j, fs] = jnp.minimum(
                        lax.bitwise_and(
                            packed, jnp.full((L,), 0xFFFF, jnp.int32)),
                        cap) + soff
                return carry2

            lax.fori_loop(0, GB, _unpack_body, 0)

            pend = pltpu.async_copy(out.at[rowsg.at[0]], msg.at[0], sem0)
            for b in range(GB):
                nxt = None
                if b + 1 < GB:
                    p = (b + 1) & 1
                    nxt = pltpu.async_copy(
                        out.at[rowsg.at[b + 1]], msg.at[p],
                        sem1 if p else sem0)
                pend.wait()
                pltpu.sync_copy(msg.at[b & 1], acc_sp.at[colsg.at[b]],
                                add=True)
                pend = nxt
            return carry

        lax.fori_loop(0, NG, _group_body, 0)

    _scale_chunk(K, "init")
    plsc.subcore_barrier()
    for i in range(K - 1, 0, -1):
        _edge_phase()
        plsc.subcore_barrier()
        _scale_chunk(i, "mid")
        plsc.subcore_barrier()
    _edge_phase()
    plsc.subcore_barrier()
    _scale_chunk(0, "final")


def kernel(x, edge_index, W1, b1, W2, b2, temp):
    h = _mlp(x, W1.T, b1[None, :], W2.T, b2[None, :])

    ei = jnp.asarray(edge_index, jnp.int32)
    packed = jnp.bitwise_or(ei[0], jnp.left_shift(ei[1], 16))
    pad = jnp.full((EP - E,), DUMMY | (DUMMY << 16), jnp.int32)
    ei3 = jnp.concatenate([packed, pad]).reshape(NS, NB, BLK)

    degrep = _deg_sc(ei3)
    deg = degrep[:N, 0]
    dis = jnp.where(deg > 0, lax.rsqrt(jnp.where(deg > 0, deg, 1.0)), 0.0)
    disL = jnp.broadcast_to(
        jnp.pad(dis, (0, NP_ - N))[:, None], (NP_, L))

    d = jnp.pad(jax.nn.relu(temp) @ _BM, (0, 16 - (K + 1)))
    dcoef = jnp.broadcast_to(d[:, None], (16, L))

    hp = jnp.pad(h, ((0, NP_ - N), (0, 0)))
    zpad = jnp.zeros((NP_, 2 * L), jnp.float32)
    hsplit = jnp.stack(
        [jnp.concatenate([hp[:, :FH], disL, disL * disL, zpad], axis=1),
         jnp.concatenate([hp[:, FH:], disL, disL * disL, zpad], axis=1)],
        axis=0)

    res = _bern_sc(ei3, hsplit, dcoef)
    prop = jnp.concatenate(
        [res[:N, :FH], res[NP_:NP_ + N, :FH]], axis=1)

    return _log_softmax(prop)
